# Initial kernel scaffold; baseline (speedup 1.0000x reference)
import jax
import jax.numpy as jnp
from jax.experimental import pallas as pl


def kernel(utter, memory, emb, W, b):
    raise NotImplementedError("write your pallas kernel here")


# speedup vs baseline: 12.1805x; 12.1805x over previous
import functools

import jax
import jax.numpy as jnp
from jax import lax
from jax.experimental import pallas as pl
from jax.experimental.pallas import tpu as pltpu
from jax.experimental.pallas import tpu_sc as plsc

HOPS = 3


def _make_segsum(n_seg: int, seg_len: int, d: int, vocab: int):
    info = plsc.get_sparse_core_info()
    nc, ns = info.num_cores, info.num_subcores
    nw = nc * ns
    assert n_seg % nw == 0
    seg_per_w = n_seg // nw
    cs = 32
    assert seg_per_w % cs == 0
    nch = seg_per_w // cs
    rows_per_chunk = cs * seg_len
    assert rows_per_chunk % 128 == 0
    ndma = rows_per_chunk // 128
    assert d % 16 == 0
    dv = d // 16

    mesh = plsc.VectorSubcoreMesh(core_axis_name="c", subcore_axis_name="s")

    @functools.partial(
        pl.kernel,
        mesh=mesh,
        out_type=jax.ShapeDtypeStruct((n_seg, d), jnp.float32),
        compiler_params=pltpu.CompilerParams(use_tc_tiling_on_sc=False),
        scratch_types=[
            pltpu.VMEM((rows_per_chunk,), jnp.int32),
            pltpu.VMEM((rows_per_chunk, d), jnp.float32),
            pltpu.VMEM((cs, d), jnp.float32),
            pltpu.SemaphoreType.DMA,
        ],
    )
    def segsum(idx_hbm, emb_hbm, out_hbm, idx_v, rows_v, out_v, sem):
        wid = lax.axis_index("s") * nc + lax.axis_index("c")

        def chunk_body(c, _):
            seg_base = wid * seg_per_w + c * cs
            ibase = seg_base * seg_len
            pltpu.sync_copy(idx_hbm.at[pl.ds(ibase, rows_per_chunk)], idx_v)
            copies = []
            for j in range(ndma):
                copies.append(
                    pltpu.async_copy(
                        emb_hbm.at[idx_v.at[pl.ds(j * 128, 128)]],
                        rows_v.at[pl.ds(j * 128, 128)],
                        sem,
                    )
                )
            for cp in copies:
                cp.wait()

            def seg_body(s, _):
                base = s * seg_len
                for v in range(dv):
                    acc = rows_v[base, pl.ds(v * 16, 16)]
                    for j in range(1, seg_len):
                        acc = acc + rows_v[base + j, pl.ds(v * 16, 16)]
                    out_v[s, pl.ds(v * 16, 16)] = acc
                return 0

            lax.fori_loop(0, cs, seg_body, 0)
            pltpu.sync_copy(out_v, out_hbm.at[pl.ds(seg_base, cs)])
            return 0

        lax.fori_loop(0, nch, chunk_body, 0)

    return segsum


def _hops_body(ms_ref, ctx_ref, wt_ref, b_ref, out_ref):
    ms = ms_ref[...]
    ctx = ctx_ref[...]
    wt = wt_ref[...]
    bv = b_ref[...]
    for _ in range(HOPS):
        attn = jnp.sum(ms * ctx[:, None, :], axis=2)
        attn = attn - jnp.max(attn, axis=1, keepdims=True)
        e = jnp.exp(attn)
        p = e / jnp.sum(e, axis=1, keepdims=True)
        stories = jnp.sum(p[:, :, None] * ms, axis=1)
        ctx = (
            jnp.dot(ctx, wt, preferred_element_type=jnp.float32) + bv + stories
        )
    out_ref[...] = ctx


def kernel(utter, memory, emb, W, b):
    bsz, seq = utter.shape
    _, mem, _ = memory.shape
    vocab, d = emb.shape

    idx_flat = jnp.concatenate(
        [memory.reshape(-1), utter.reshape(-1)]
    ).astype(jnp.int32)

    n_seg = bsz * mem + bsz
    sums = _make_segsum(n_seg, seq, d, vocab)(idx_flat, emb)

    ms = sums[: bsz * mem].reshape(bsz, mem, d)
    ctx0 = sums[bsz * mem :]

    bb = 128
    grid = (bsz // bb,)
    out = pl.pallas_call(
        _hops_body,
        grid=grid,
        in_specs=[
            pl.BlockSpec((bb, mem, d), lambda i: (i, 0, 0)),
            pl.BlockSpec((bb, d), lambda i: (i, 0)),
            pl.BlockSpec((d, d), lambda i: (0, 0)),
            pl.BlockSpec((1, d), lambda i: (0, 0)),
        ],
        out_specs=pl.BlockSpec((bb, d), lambda i: (i, 0)),
        out_shape=jax.ShapeDtypeStruct((bsz, d), jnp.float32),
    )(ms, ctx0, W.T, b.reshape(1, d))
    return out


# speedup vs baseline: 15.8064x; 1.2977x over previous
import functools

import jax
import jax.numpy as jnp
from jax import lax
from jax.experimental import pallas as pl
from jax.experimental.pallas import tpu as pltpu
from jax.experimental.pallas import tpu_sc as plsc

HOPS = 3


def _make_segsum(n_seg: int, seg_len: int, d: int, vocab: int):
    info = plsc.get_sparse_core_info()
    nc, ns = info.num_cores, info.num_subcores
    nw = nc * ns
    assert n_seg % nw == 0
    seg_per_w = n_seg // nw
    cs = 32
    assert seg_per_w % cs == 0
    nch = seg_per_w // cs
    rows_per_chunk = cs * seg_len
    assert rows_per_chunk % 128 == 0
    ndma = rows_per_chunk // 128
    assert d % 16 == 0
    dv = d // 16

    mesh = plsc.VectorSubcoreMesh(core_axis_name="c", subcore_axis_name="s")

    @functools.partial(
        pl.kernel,
        mesh=mesh,
        out_type=jax.ShapeDtypeStruct((n_seg, d), jnp.float32),
        compiler_params=pltpu.CompilerParams(use_tc_tiling_on_sc=False),
        scratch_types=[
            pltpu.VMEM((2 * rows_per_chunk,), jnp.int32),
            pltpu.VMEM((2 * rows_per_chunk, d), jnp.float32),
            pltpu.VMEM((seg_per_w, d), jnp.float32),
            pltpu.SemaphoreType.DMA,
            pltpu.SemaphoreType.DMA,
        ],
    )
    def segsum(idx_hbm, emb_hbm, out_hbm, idx_v, rows_v, out_v, sem_i, sem_g):
        wid = lax.axis_index("s") * nc + lax.axis_index("c")
        wbase = wid * seg_per_w
        ibase0 = wbase * seg_len
        rpc = rows_per_chunk

        def fire_idx(c):
            buf = lax.rem(c, 2) * rpc
            pltpu.async_copy(
                idx_hbm.at[pl.ds(ibase0 + c * rpc, rpc)],
                idx_v.at[pl.ds(buf, rpc)],
                sem_i,
            )

        def wait_idx():
            pltpu.make_async_copy(
                idx_hbm.at[pl.ds(0, rpc)], idx_v.at[pl.ds(0, rpc)], sem_i
            ).wait()

        def fire_gathers(c):
            buf = lax.rem(c, 2) * rpc
            for j in range(ndma):
                pltpu.async_copy(
                    emb_hbm.at[idx_v.at[pl.ds(buf + j * 128, 128)]],
                    rows_v.at[pl.ds(buf + j * 128, 128)],
                    sem_g,
                )

        def wait_gathers():
            for j in range(ndma):
                pltpu.make_async_copy(
                    emb_hbm.at[pl.ds(0, 128)],
                    rows_v.at[pl.ds(j * 128, 128)],
                    sem_g,
                ).wait()

        fire_idx(0)
        fire_idx(1)
        wait_idx()
        fire_gathers(0)

        def chunk_body(c, _):
            wait_idx()
            fire_gathers(jnp.minimum(c + 1, nch - 1))
            wait_gathers()
            fire_idx(jnp.minimum(c + 2, nch - 1))
            rbase = lax.rem(c, 2) * rpc

            def seg_body(s, _):
                base = rbase + s * seg_len
                for v in range(dv):
                    acc = rows_v[base, pl.ds(v * 16, 16)]
                    for j in range(1, seg_len):
                        acc = acc + rows_v[base + j, pl.ds(v * 16, 16)]
                    out_v[c * cs + s, pl.ds(v * 16, 16)] = acc
                return 0

            lax.fori_loop(0, cs, seg_body, 0)
            return 0

        lax.fori_loop(0, nch, chunk_body, 0)
        wait_idx()
        wait_gathers()
        pltpu.sync_copy(out_v, out_hbm.at[pl.ds(wbase, seg_per_w)])

    return segsum


def _hops_body(ms_ref, ctx_ref, wt_ref, b_ref, out_ref):
    ms = ms_ref[...]
    ctx = ctx_ref[...]
    wt = wt_ref[...]
    bv = b_ref[...]
    for _ in range(HOPS):
        attn = jnp.sum(ms * ctx[:, None, :], axis=2)
        attn = attn - jnp.max(attn, axis=1, keepdims=True)
        e = jnp.exp(attn)
        p = e / jnp.sum(e, axis=1, keepdims=True)
        stories = jnp.sum(p[:, :, None] * ms, axis=1)
        ctx = (
            jnp.dot(ctx, wt, preferred_element_type=jnp.float32) + bv + stories
        )
    out_ref[...] = ctx


def kernel(utter, memory, emb, W, b):
    bsz, seq = utter.shape
    _, mem, _ = memory.shape
    vocab, d = emb.shape

    idx_flat = jnp.concatenate(
        [memory.reshape(-1), utter.reshape(-1)]
    ).astype(jnp.int32)

    n_seg = bsz * mem + bsz
    sums = _make_segsum(n_seg, seq, d, vocab)(idx_flat, emb)

    ms = sums[: bsz * mem].reshape(bsz, mem, d)
    ctx0 = sums[bsz * mem :]

    bb = 128
    grid = (bsz // bb,)
    out = pl.pallas_call(
        _hops_body,
        grid=grid,
        in_specs=[
            pl.BlockSpec((bb, mem, d), lambda i: (i, 0, 0)),
            pl.BlockSpec((bb, d), lambda i: (i, 0)),
            pl.BlockSpec((d, d), lambda i: (0, 0)),
            pl.BlockSpec((1, d), lambda i: (0, 0)),
        ],
        out_specs=pl.BlockSpec((bb, d), lambda i: (i, 0)),
        out_shape=jax.ShapeDtypeStruct((bsz, d), jnp.float32),
    )(ms, ctx0, W.T, b.reshape(1, d))
    return out


# speedup vs baseline: 18.7881x; 1.1886x over previous
import functools

import jax
import jax.numpy as jnp
from jax import lax
from jax.experimental import pallas as pl
from jax.experimental.pallas import tpu as pltpu
from jax.experimental.pallas import tpu_sc as plsc

HOPS = 3


def _make_segsum(n_seg: int, seg_len: int, d: int, vocab: int):
    info = plsc.get_sparse_core_info()
    nc, ns = info.num_cores, info.num_subcores
    nw = nc * ns
    assert n_seg % nw == 0
    seg_per_w = n_seg // nw
    cs = 32
    assert seg_per_w % cs == 0
    nch = seg_per_w // cs
    rows_per_chunk = cs * seg_len
    assert rows_per_chunk % 128 == 0
    ndma = rows_per_chunk // 128
    assert d % 16 == 0
    dv = d // 16

    mesh = plsc.VectorSubcoreMesh(core_axis_name="c", subcore_axis_name="s")

    @functools.partial(
        pl.kernel,
        mesh=mesh,
        out_type=jax.ShapeDtypeStruct((n_seg, d), jnp.float32),
        compiler_params=pltpu.CompilerParams(use_tc_tiling_on_sc=False),
        scratch_types=[
            pltpu.VMEM((2 * rows_per_chunk,), jnp.int32),
            pltpu.VMEM((2 * rows_per_chunk, d), jnp.float32),
            pltpu.VMEM((seg_per_w, d), jnp.float32),
            pltpu.SemaphoreType.DMA,
            pltpu.SemaphoreType.DMA,
        ],
    )
    def segsum(idx_hbm, emb_hbm, out_hbm, idx_v, rows_v, out_v, sem_i, sem_g):
        wid = lax.axis_index("s") * nc + lax.axis_index("c")
        wbase = wid * seg_per_w
        ibase0 = wbase * seg_len
        rpc = rows_per_chunk

        def fire_idx(c):
            buf = lax.rem(c, 2) * rpc
            pltpu.async_copy(
                idx_hbm.at[pl.ds(ibase0 + c * rpc, rpc)],
                idx_v.at[pl.ds(buf, rpc)],
                sem_i,
            )

        def wait_idx():
            pltpu.make_async_copy(
                idx_hbm.at[pl.ds(0, rpc)], idx_v.at[pl.ds(0, rpc)], sem_i
            ).wait()

        def fire_gathers(c):
            buf = lax.rem(c, 2) * rpc
            for j in range(ndma):
                pltpu.async_copy(
                    emb_hbm.at[idx_v.at[pl.ds(buf + j * 128, 128)]],
                    rows_v.at[pl.ds(buf + j * 128, 128)],
                    sem_g,
                )

        def wait_gathers():
            for j in range(ndma):
                pltpu.make_async_copy(
                    emb_hbm.at[pl.ds(0, 128)],
                    rows_v.at[pl.ds(j * 128, 128)],
                    sem_g,
                ).wait()

        fire_idx(0)
        fire_idx(1)
        wait_idx()
        fire_gathers(0)

        def chunk_body(c, _):
            wait_idx()
            fire_gathers(jnp.minimum(c + 1, nch - 1))
            wait_gathers()
            fire_idx(jnp.minimum(c + 2, nch - 1))
            rbase = lax.rem(c, 2) * rpc

            def seg_body(s, _):
                base = rbase + s * seg_len
                for v in range(dv):
                    acc = rows_v[base, pl.ds(v * 16, 16)]
                    for j in range(1, seg_len):
                        acc = acc + rows_v[base + j, pl.ds(v * 16, 16)]
                    out_v[c * cs + s, pl.ds(v * 16, 16)] = acc
                return 0

            lax.fori_loop(0, cs, seg_body, 0)
            return 0

        lax.fori_loop(0, nch, chunk_body, 0)
        wait_idx()
        wait_gathers()
        pltpu.sync_copy(out_v, out_hbm.at[pl.ds(wbase, seg_per_w)])

    return segsum


def _hops_body(ms_ref, ctx_ref, w_ref, b_ref, out_ref):
    ms = ms_ref[...]
    ctx = ctx_ref[...]
    w = w_ref[...]
    bv = b_ref[...]
    for _ in range(HOPS):
        attn = jnp.sum(ms * ctx[None, :, :], axis=1)
        attn = attn - jnp.max(attn, axis=0, keepdims=True)
        e = jnp.exp(attn)
        p = e / jnp.sum(e, axis=0, keepdims=True)
        stories = jnp.sum(p[:, None, :] * ms, axis=0)
        ctx = (
            jnp.dot(w, ctx, preferred_element_type=jnp.float32) + bv + stories
        )
    out_ref[...] = ctx


def kernel(utter, memory, emb, W, b):
    bsz, seq = utter.shape
    _, mem, _ = memory.shape
    vocab, d = emb.shape

    idx_flat = jnp.concatenate(
        [memory.reshape(-1), utter.reshape(-1)]
    ).astype(jnp.int32)

    n_seg = bsz * mem + bsz
    sums = _make_segsum(n_seg, seq, d, vocab)(idx_flat, emb)

    ms_t = jnp.transpose(sums[: bsz * mem].reshape(bsz, mem, d), (1, 2, 0))
    ctx0_t = sums[bsz * mem :].T

    bb = 128
    grid = (bsz // bb,)
    out_t = pl.pallas_call(
        _hops_body,
        grid=grid,
        in_specs=[
            pl.BlockSpec((mem, d, bb), lambda i: (0, 0, i)),
            pl.BlockSpec((d, bb), lambda i: (0, i)),
            pl.BlockSpec((d, d), lambda i: (0, 0)),
            pl.BlockSpec((d, 1), lambda i: (0, 0)),
        ],
        out_specs=pl.BlockSpec((d, bb), lambda i: (0, i)),
        out_shape=jax.ShapeDtypeStruct((d, bsz), jnp.float32),
    )(ms_t, ctx0_t, W, b.reshape(d, 1))
    return out_t.T


# speedup vs baseline: 22.8049x; 1.2138x over previous
import functools

import jax
import jax.numpy as jnp
from jax import lax
from jax.experimental import pallas as pl
from jax.experimental.pallas import tpu as pltpu
from jax.experimental.pallas import tpu_sc as plsc

HOPS = 3


def _make_segsum(n_mem_seg: int, n_utt_seg: int, seg_len: int, d: int):
    info = plsc.get_sparse_core_info()
    nc, ns = info.num_cores, info.num_subcores
    nw = nc * ns
    assert n_mem_seg % nw == 0 and n_utt_seg % nw == 0
    mseg_w = n_mem_seg // nw
    useg_w = n_utt_seg // nw
    cs = 32
    assert mseg_w % cs == 0 and useg_w == cs
    nch_m = mseg_w // cs
    nch = nch_m + 1
    rpc = cs * seg_len
    assert rpc % 128 == 0
    ndma = rpc // 128
    assert d % 16 == 0
    dv = d // 16

    mesh = plsc.VectorSubcoreMesh(core_axis_name="c", subcore_axis_name="s")

    @functools.partial(
        pl.kernel,
        mesh=mesh,
        out_type=[
            jax.ShapeDtypeStruct((n_mem_seg, d), jnp.float32),
            jax.ShapeDtypeStruct((n_utt_seg, d), jnp.float32),
        ],
        compiler_params=pltpu.CompilerParams(use_tc_tiling_on_sc=False),
        scratch_types=[
            pltpu.VMEM((2 * rpc,), jnp.int32),
            pltpu.VMEM((2 * rpc, d), jnp.float32),
            pltpu.VMEM((mseg_w + useg_w, d), jnp.float32),
            pltpu.SemaphoreType.DMA,
            pltpu.SemaphoreType.DMA,
        ],
    )
    def segsum(midx_hbm, uidx_hbm, emb_hbm, oms_hbm, octx_hbm,
               idx_v, rows_v, out_v, sem_i, sem_g):
        wid = lax.axis_index("s") * nc + lax.axis_index("c")
        mbase = wid * mseg_w * seg_len
        ubase = wid * useg_w * seg_len

        def fire_idx(c):
            buf = lax.rem(c, 2) * rpc
            dst = idx_v.at[pl.ds(buf, rpc)]

            @pl.when(c < nch_m)
            def _():
                pltpu.async_copy(
                    midx_hbm.at[pl.ds(mbase + c * rpc, rpc)], dst, sem_i
                )

            @pl.when(c >= nch_m)
            def _():
                pltpu.async_copy(uidx_hbm.at[pl.ds(ubase, rpc)], dst, sem_i)

        def wait_idx():
            pltpu.make_async_copy(
                uidx_hbm.at[pl.ds(0, rpc)], idx_v.at[pl.ds(0, rpc)], sem_i
            ).wait()

        def fire_gathers(c):
            buf = lax.rem(c, 2) * rpc
            for j in range(ndma):
                pltpu.async_copy(
                    emb_hbm.at[idx_v.at[pl.ds(buf + j * 128, 128)]],
                    rows_v.at[pl.ds(buf + j * 128, 128)],
                    sem_g,
                )

        def wait_gathers():
            for j in range(ndma):
                pltpu.make_async_copy(
                    emb_hbm.at[pl.ds(0, 128)],
                    rows_v.at[pl.ds(j * 128, 128)],
                    sem_g,
                ).wait()

        fire_idx(0)
        fire_idx(1)
        wait_idx()
        fire_gathers(0)

        def chunk_body(c, _):
            wait_idx()
            fire_gathers(jnp.minimum(c + 1, nch - 1))
            wait_gathers()
            fire_idx(jnp.minimum(c + 2, nch - 1))
            rbase = lax.rem(c, 2) * rpc

            def seg_body(s, _):
                base = rbase + s * seg_len
                for v in range(dv):
                    acc = rows_v[base, pl.ds(v * 16, 16)]
                    for j in range(1, seg_len):
                        acc = acc + rows_v[base + j, pl.ds(v * 16, 16)]
                    out_v[c * cs + s, pl.ds(v * 16, 16)] = acc
                return 0

            lax.fori_loop(0, cs, seg_body, 0)
            return 0

        lax.fori_loop(0, nch, chunk_body, 0)
        wait_idx()
        wait_gathers()
        pltpu.sync_copy(
            out_v.at[pl.ds(0, mseg_w)], oms_hbm.at[pl.ds(wid * mseg_w, mseg_w)]
        )
        pltpu.sync_copy(
            out_v.at[pl.ds(mseg_w, useg_w)],
            octx_hbm.at[pl.ds(wid * useg_w, useg_w)],
        )

    return segsum


def _hops_body(ms_ref, ctx_ref, w_ref, b_ref, out_ref):
    mem = ms_ref.shape[0]
    ms = jnp.transpose(ms_ref[...], (0, 2, 1))
    ctx = ctx_ref[...].T
    w = w_ref[...]
    bv = b_ref[...]
    for _ in range(HOPS):
        attn = jnp.sum(ms * ctx[None, :, :], axis=1)
        attn = attn - jnp.max(attn, axis=0, keepdims=True)
        e = jnp.exp(attn)
        p = e / jnp.sum(e, axis=0, keepdims=True)
        stories = jnp.sum(p[:, None, :] * ms, axis=0)
        ctx = (
            jnp.dot(w, ctx, preferred_element_type=jnp.float32) + bv + stories
        )
    out_ref[...] = ctx.T


def kernel(utter, memory, emb, W, b):
    bsz, seq = utter.shape
    _, mem, _ = memory.shape
    _, d = emb.shape

    midx = jnp.transpose(memory, (1, 0, 2)).reshape(-1).astype(jnp.int32)
    uidx = utter.reshape(-1).astype(jnp.int32)

    ms2, ctx0 = _make_segsum(bsz * mem, bsz, seq, d)(midx, uidx, emb)

    bb = 128
    grid = (bsz // bb,)
    out = pl.pallas_call(
        _hops_body,
        grid=grid,
        in_specs=[
            pl.BlockSpec((mem, bb, d), lambda i: (0, i, 0)),
            pl.BlockSpec((bb, d), lambda i: (i, 0)),
            pl.BlockSpec((d, d), lambda i: (0, 0)),
            pl.BlockSpec((d, 1), lambda i: (0, 0)),
        ],
        out_specs=pl.BlockSpec((bb, d), lambda i: (i, 0)),
        out_shape=jax.ShapeDtypeStruct((bsz, d), jnp.float32),
    )(ms2.reshape(mem, bsz, d), ctx0, W, b.reshape(d, 1))
    return out
